# baseline (speedup 1.0000x reference)
import functools

import jax
import jax.numpy as jnp
from jax import lax
from jax.experimental import pallas as pl
from jax.experimental.pallas import tpu as pltpu
from jax.experimental.pallas import tpu_sc as plsc

_C = 1000
_D = 128
_B = 16384
_NC, _NS = 2, 16
_NW = _NC * _NS
_BW = _B // _NW
_SUB = 256
_HIST = 1024
_LPC = _B // _NS

_vmesh = plsc.VectorSubcoreMesh(core_axis_name="c", subcore_axis_name="s")


def _sc_body(y_hbm, centers_hbm, g_hbm, w_hbm,
             hist_sh, idxA, ones_v, zer_v, idx_v, cnt_v, rows_v, sem):
    c = lax.axis_index("c")
    s = lax.axis_index("s")
    wid = s * _NC + c

    @pl.loop(0, _LPC, step=16)
    def _(i):
        ones_v[pl.ds(i, 16)] = jnp.full((16,), 1.0, jnp.float32)

    @pl.loop(0, _HIST, step=16)
    def _(i):
        zer_v[pl.ds(i, 16)] = jnp.zeros((16,), jnp.float32)

    @pl.when(s == 0)
    def _():
        pltpu.sync_copy(zer_v, hist_sh)

    plsc.subcore_barrier()

    pltpu.sync_copy(y_hbm.at[pl.ds(s * _LPC, _LPC)], idxA)
    pltpu.sync_copy(ones_v, hist_sh.at[idxA], add=True)
    plsc.subcore_barrier()

    base = wid * _BW
    pltpu.sync_copy(y_hbm.at[pl.ds(base, _BW)], idx_v)
    pltpu.sync_copy(hist_sh.at[idx_v], cnt_v)

    @pl.loop(0, _BW, step=16)
    def _(i):
        cnt_v[pl.ds(i, 16)] = 1.0 / (cnt_v[pl.ds(i, 16)] + 1.0)

    pltpu.sync_copy(cnt_v, w_hbm.at[pl.ds(base, _BW)])

    for k in range(_BW // _SUB):
        pltpu.async_copy(
            centers_hbm.at[idx_v.at[pl.ds(k * _SUB, _SUB)]], rows_v, sem
        ).wait()
        pltpu.sync_copy(rows_v, g_hbm.at[pl.ds(base + k * _SUB, _SUB)])


_sc_gather = functools.partial(
    pl.kernel,
    out_type=(
        jax.ShapeDtypeStruct((_B, _D), jnp.float32),
        jax.ShapeDtypeStruct((_B,), jnp.float32),
    ),
    mesh=_vmesh,
    scratch_types=[
        pltpu.VMEM_SHARED((_HIST,), jnp.float32),
        pltpu.VMEM((_LPC,), jnp.int32),
        pltpu.VMEM((_LPC,), jnp.float32),
        pltpu.VMEM((_HIST,), jnp.float32),
        pltpu.VMEM((_BW,), jnp.int32),
        pltpu.VMEM((_BW,), jnp.float32),
        pltpu.VMEM((_SUB, _D), jnp.float32),
        pltpu.SemaphoreType.DMA,
    ],
)(_sc_body)

_RB = 1024


def _tc_body(feat_ref, g_ref, w_ref, out_ref):
    i = pl.program_id(0)

    @pl.when(i == 0)
    def _():
        out_ref[...] = jnp.zeros_like(out_ref)

    diff = feat_ref[...] - g_ref[...]
    d2 = jnp.sum(diff * diff, axis=1)
    out_ref[...] += jnp.reshape(0.5 * jnp.sum(d2 * w_ref[0, 0]), (1, 1))


def kernel(y, feat, centers):
    g, w = _sc_gather(y.astype(jnp.int32), centers)
    w3 = w.reshape(_B // _RB, 1, _RB)
    out = pl.pallas_call(
        _tc_body,
        grid=(_B // _RB,),
        in_specs=[
            pl.BlockSpec((_RB, _D), lambda i: (i, 0)),
            pl.BlockSpec((_RB, _D), lambda i: (i, 0)),
            pl.BlockSpec((1, 1, _RB), lambda i: (i, 0, 0)),
        ],
        out_specs=pl.BlockSpec((1, 1), lambda i: (0, 0)),
        out_shape=jax.ShapeDtypeStruct((1, 1), jnp.float32),
    )(feat, g, w3)
    return out[0, 0]


# speedup vs baseline: 1.6887x; 1.6887x over previous
import jax
import jax.numpy as jnp
from jax import lax
from jax.experimental import pallas as pl
from jax.experimental.pallas import tpu as pltpu

_NUM_CLASSES = 1000
_FEAT = 128
_BATCH = 16384
_CPAD = 1024
_BB = 512
_XW = _FEAT + 2


def _body(y_ref, feat_ref, centers_ref, out_ref, acc_ref):
    i = pl.program_id(0)
    nsteps = pl.num_programs(0)

    @pl.when(i == 0)
    def _init():
        acc_ref[...] = jnp.zeros_like(acc_ref)

    yb = y_ref[0]
    fb = feat_ref[...]

    cls = lax.broadcasted_iota(jnp.int32, (_CPAD, _BB), 0)
    ohT = jnp.where(cls == jnp.broadcast_to(yb, (_CPAD, _BB)),
                    1.0, 0.0).astype(jnp.bfloat16)

    q = jnp.sum(fb * fb, axis=1, keepdims=True)
    x = jnp.concatenate(
        [fb.astype(jnp.bfloat16), q.astype(jnp.bfloat16),
         jnp.ones((_BB, 1), jnp.bfloat16)], axis=1)

    acc_ref[...] += jnp.dot(ohT, x, preferred_element_type=jnp.float32)

    @pl.when(i == nsteps - 1)
    def _fini():
        C = centers_ref[...]
        m = acc_ref[:, :_FEAT]
        S2 = acc_ref[:, _FEAT]
        n = acc_ref[:, _FEAT + 1]
        z = jnp.sum(C * C, axis=1)
        d = jnp.sum(m * C, axis=1)
        num = S2 - 2.0 * d + n * z
        out_ref[...] = jnp.reshape(0.5 * jnp.sum(num / (n + 1.0)), (1, 1))


def kernel(y, feat, centers):
    y3 = y.astype(jnp.int32).reshape(_BATCH // _BB, 1, _BB)
    cpad = jnp.pad(centers, ((0, _CPAD - _NUM_CLASSES), (0, 0)))
    out = pl.pallas_call(
        _body,
        grid=(_BATCH // _BB,),
        in_specs=[
            pl.BlockSpec((1, 1, _BB), lambda i: (i, 0, 0)),
            pl.BlockSpec((_BB, _FEAT), lambda i: (i, 0)),
            pl.BlockSpec((_CPAD, _FEAT), lambda i: (0, 0)),
        ],
        out_specs=pl.BlockSpec((1, 1), lambda i: (0, 0)),
        out_shape=jax.ShapeDtypeStruct((1, 1), jnp.float32),
        scratch_shapes=[
            pltpu.VMEM((_CPAD, _XW), jnp.float32),
        ],
    )(y3, feat, cpad)
    return out[0, 0]


# speedup vs baseline: 2.4474x; 1.4493x over previous
import jax
import jax.numpy as jnp
from jax import lax
from jax.experimental import pallas as pl
from jax.experimental.pallas import tpu as pltpu

_NUM_CLASSES = 1000
_FEAT = 128
_BATCH = 16384
_CPAD = 1024
_BB = 1024
_XW = _FEAT + 2


def _body(y_ref, feat_ref, centers_ref, out_ref, acc_ref):
    i = pl.program_id(0)
    nsteps = pl.num_programs(0)

    @pl.when(i == 0)
    def _init():
        acc_ref[...] = jnp.zeros_like(acc_ref)

    yb = y_ref[0]
    fb = feat_ref[...]

    cls = lax.broadcasted_iota(jnp.int32, (_CPAD, _BB), 0)
    ohT = jnp.where(cls == jnp.broadcast_to(yb, (_CPAD, _BB)),
                    1.0, 0.0).astype(jnp.bfloat16)

    q = jnp.sum(fb * fb, axis=1, keepdims=True)
    x = jnp.concatenate(
        [fb.astype(jnp.bfloat16), q.astype(jnp.bfloat16),
         jnp.ones((_BB, 1), jnp.bfloat16)], axis=1)

    acc_ref[...] += jnp.dot(ohT, x, preferred_element_type=jnp.float32)

    @pl.when(i == nsteps - 1)
    def _fini():
        C = centers_ref[...]
        m = acc_ref[:, :_FEAT]
        S2 = acc_ref[:, _FEAT]
        n = acc_ref[:, _FEAT + 1]
        z = jnp.sum(C * C, axis=1)
        d = jnp.sum(m * C, axis=1)
        num = S2 - 2.0 * d + n * z
        out_ref[...] = jnp.reshape(0.5 * jnp.sum(num / (n + 1.0)), (1, 1))


def kernel(y, feat, centers):
    y3 = y.astype(jnp.int32).reshape(_BATCH // _BB, 1, _BB)
    cpad = jnp.pad(centers, ((0, _CPAD - _NUM_CLASSES), (0, 0)))
    out = pl.pallas_call(
        _body,
        grid=(_BATCH // _BB,),
        in_specs=[
            pl.BlockSpec((1, 1, _BB), lambda i: (i, 0, 0)),
            pl.BlockSpec((_BB, _FEAT), lambda i: (i, 0)),
            pl.BlockSpec((_CPAD, _FEAT), lambda i: (0, 0)),
        ],
        out_specs=pl.BlockSpec((1, 1), lambda i: (0, 0)),
        out_shape=jax.ShapeDtypeStruct((1, 1), jnp.float32),
        scratch_shapes=[
            pltpu.VMEM((_CPAD, _XW), jnp.float32),
        ],
    )(y3, feat, cpad)
    return out[0, 0]


# speedup vs baseline: 3.0575x; 1.2492x over previous
import jax
import jax.numpy as jnp
from jax import lax
from jax.experimental import pallas as pl
from jax.experimental.pallas import tpu as pltpu

_NUM_CLASSES = 1000
_FEAT = 128
_BATCH = 16384
_CPAD = 1024
_BB = 2048
_XW = _FEAT + 2


def _body(y_ref, feat_ref, centers_ref, out_ref, acc_ref):
    i = pl.program_id(0)
    nsteps = pl.num_programs(0)

    @pl.when(i == 0)
    def _init():
        acc_ref[...] = jnp.zeros_like(acc_ref)

    yb = y_ref[0]
    fb = feat_ref[...]

    cls = lax.broadcasted_iota(jnp.int32, (_CPAD, _BB), 0)
    ohT = jnp.where(cls == jnp.broadcast_to(yb, (_CPAD, _BB)),
                    1.0, 0.0).astype(jnp.bfloat16)

    q = jnp.sum(fb * fb, axis=1, keepdims=True)
    x = jnp.concatenate(
        [fb.astype(jnp.bfloat16), q.astype(jnp.bfloat16),
         jnp.ones((_BB, 1), jnp.bfloat16)], axis=1)

    acc_ref[...] += jnp.dot(ohT, x, preferred_element_type=jnp.float32)

    @pl.when(i == nsteps - 1)
    def _fini():
        C = centers_ref[...]
        m = acc_ref[:, :_FEAT]
        S2 = acc_ref[:, _FEAT]
        n = acc_ref[:, _FEAT + 1]
        z = jnp.sum(C * C, axis=1)
        d = jnp.sum(m * C, axis=1)
        num = S2 - 2.0 * d + n * z
        out_ref[...] = jnp.reshape(0.5 * jnp.sum(num / (n + 1.0)), (1, 1))


def kernel(y, feat, centers):
    y3 = y.astype(jnp.int32).reshape(_BATCH // _BB, 1, _BB)
    cpad = jnp.pad(centers, ((0, _CPAD - _NUM_CLASSES), (0, 0)))
    out = pl.pallas_call(
        _body,
        grid=(_BATCH // _BB,),
        in_specs=[
            pl.BlockSpec((1, 1, _BB), lambda i: (i, 0, 0)),
            pl.BlockSpec((_BB, _FEAT), lambda i: (i, 0)),
            pl.BlockSpec((_CPAD, _FEAT), lambda i: (0, 0)),
        ],
        out_specs=pl.BlockSpec((1, 1), lambda i: (0, 0)),
        out_shape=jax.ShapeDtypeStruct((1, 1), jnp.float32),
        scratch_shapes=[
            pltpu.VMEM((_CPAD, _XW), jnp.float32),
        ],
    )(y3, feat, cpad)
    return out[0, 0]


# speedup vs baseline: 3.2318x; 1.0570x over previous
import jax
import jax.numpy as jnp
from jax import lax
from jax.experimental import pallas as pl
from jax.experimental.pallas import tpu as pltpu

_NUM_CLASSES = 1000
_FEAT = 128
_BATCH = 16384
_CPAD = 1024
_BB = 4096
_XW = _FEAT + 2


def _body(y_ref, feat_ref, centers_ref, out_ref, acc_ref):
    i = pl.program_id(0)
    nsteps = pl.num_programs(0)

    @pl.when(i == 0)
    def _init():
        acc_ref[...] = jnp.zeros_like(acc_ref)

    yb = y_ref[0]
    fb = feat_ref[...]

    cls = lax.broadcasted_iota(jnp.int32, (_CPAD, _BB), 0)
    ohT = jnp.where(cls == jnp.broadcast_to(yb, (_CPAD, _BB)),
                    1.0, 0.0).astype(jnp.bfloat16)

    q = jnp.sum(fb * fb, axis=1, keepdims=True)
    x = jnp.concatenate(
        [fb.astype(jnp.bfloat16), q.astype(jnp.bfloat16),
         jnp.ones((_BB, 1), jnp.bfloat16)], axis=1)

    acc_ref[...] += jnp.dot(ohT, x, preferred_element_type=jnp.float32)

    @pl.when(i == nsteps - 1)
    def _fini():
        C = centers_ref[...]
        m = acc_ref[:, :_FEAT]
        S2 = acc_ref[:, _FEAT]
        n = acc_ref[:, _FEAT + 1]
        z = jnp.sum(C * C, axis=1)
        d = jnp.sum(m * C, axis=1)
        num = S2 - 2.0 * d + n * z
        out_ref[...] = jnp.reshape(0.5 * jnp.sum(num / (n + 1.0)), (1, 1))


def kernel(y, feat, centers):
    y3 = y.astype(jnp.int32).reshape(_BATCH // _BB, 1, _BB)
    cpad = jnp.pad(centers, ((0, _CPAD - _NUM_CLASSES), (0, 0)))
    out = pl.pallas_call(
        _body,
        grid=(_BATCH // _BB,),
        in_specs=[
            pl.BlockSpec((1, 1, _BB), lambda i: (i, 0, 0)),
            pl.BlockSpec((_BB, _FEAT), lambda i: (i, 0)),
            pl.BlockSpec((_CPAD, _FEAT), lambda i: (0, 0)),
        ],
        out_specs=pl.BlockSpec((1, 1), lambda i: (0, 0)),
        out_shape=jax.ShapeDtypeStruct((1, 1), jnp.float32),
        scratch_shapes=[
            pltpu.VMEM((_CPAD, _XW), jnp.float32),
        ],
    )(y3, feat, cpad)
    return out[0, 0]
